# baseline (speedup 1.0000x reference)
import jax
import jax.numpy as jnp
from jax import lax
from jax.experimental import pallas as pl
from jax.experimental.pallas import tpu as pltpu
from jax.experimental.pallas import tpu_sc as plsc

B = 16384
F = 32
NUM_CORES = 2
NUM_SUBCORES = 16
NW = NUM_CORES * NUM_SUBCORES
BPW = B // NW
CHUNK = 128
NCHUNK = BPW // CHUNK
L = 16
GROUPS = BPW // L


def _cf_body(user_hbm, item_hbm, uf_hbm, if_hbm, out_hbm,
             uidx_v, iidx_v, urows_v, irows_v, out_v, sem):
    wid = lax.axis_index("s") * NUM_CORES + lax.axis_index("c")
    base = wid * BPW

    pltpu.sync_copy(user_hbm.at[pl.ds(base, BPW)], uidx_v)
    pltpu.sync_copy(item_hbm.at[pl.ds(base, BPW)], iidx_v)

    copies = []
    for j in range(NCHUNK):
        sl = pl.ds(j * CHUNK, CHUNK)
        copies.append(pltpu.async_copy(uf_hbm.at[uidx_v.at[sl]], urows_v.at[sl], sem))
        copies.append(pltpu.async_copy(if_hbm.at[iidx_v.at[sl]], irows_v.at[sl], sem))
    for c in copies:
        c.wait()

    def group(g, carry):
        rows = g * L + lax.iota(jnp.int32, L)
        acc = jnp.zeros((L,), jnp.float32)
        for f in range(F):
            col = jnp.full((L,), f, jnp.int32)
            uvec = plsc.load_gather(urows_v, [rows, col])
            vvec = plsc.load_gather(irows_v, [rows, col])
            acc = acc + uvec * vvec
        out_v[pl.ds(g * L, L)] = acc
        return carry

    lax.fori_loop(0, GROUPS, group, 0)

    pltpu.sync_copy(out_v, out_hbm.at[pl.ds(base, BPW)])


@jax.jit
def _cf(user, item, user_factors, item_factors):
    kern = pl.kernel(
        _cf_body,
        out_type=jax.ShapeDtypeStruct((B,), jnp.float32),
        mesh=plsc.VectorSubcoreMesh(core_axis_name="c", subcore_axis_name="s"),
        compiler_params=pltpu.CompilerParams(
            needs_layout_passes=False, use_tc_tiling_on_sc=False),
        scratch_types=[
            pltpu.VMEM((BPW,), jnp.int32),
            pltpu.VMEM((BPW,), jnp.int32),
            pltpu.VMEM((BPW, F), jnp.float32),
            pltpu.VMEM((BPW, F), jnp.float32),
            pltpu.VMEM((BPW,), jnp.float32),
            pltpu.SemaphoreType.DMA,
        ],
    )
    return kern(user, item, user_factors, item_factors)


def kernel(user, item, user_factors, item_factors):
    return _cf(user.astype(jnp.int32), item.astype(jnp.int32),
               user_factors, item_factors)


# speedup vs baseline: 3.1419x; 3.1419x over previous
import jax
import jax.numpy as jnp
from jax import lax
from jax.experimental import pallas as pl
from jax.experimental.pallas import tpu as pltpu
from jax.experimental.pallas import tpu_sc as plsc

B = 16384
F = 32
NUM_CORES = 2
NUM_SUBCORES = 16
NW = NUM_CORES * NUM_SUBCORES
BPW = B // NW
L = 16
NITER = BPW // L


def _iota():
    return lax.iota(jnp.int32, L)


def _perm(x, idx):
    return jnp.take(x, idx, axis=0)


def _cf_body(ut_hbm, vt_hbm, user_hbm, item_hbm, out_hbm,
             uidx_v, iidx_v, ubuf_v, vbuf_v, out_v, sem):
    wid = lax.axis_index("s") * NUM_CORES + lax.axis_index("c")
    base = wid * BPW

    pltpu.sync_copy(user_hbm.at[pl.ds(base, BPW)], uidx_v)
    pltpu.sync_copy(item_hbm.at[pl.ds(base, BPW)], iidx_v)

    iota = _iota()
    lane_mod4 = iota & 3
    row_div4 = iota >> 2

    def step(i, carry):
        uvec = uidx_v[pl.ds(i * L, L)]
        ivec = iidx_v[pl.ds(i * L, L)]
        lu = uvec & 127
        lv = ivec & 127
        out16 = jnp.zeros((L,), jnp.float32)

        for rnd in range(4):
            slot = rnd % 2
            copies = []
            for j in range(4):
                us = uvec[4 * rnd + j] & -128
                copies.append(pltpu.async_copy(
                    ut_hbm.at[:, pl.ds(pl.multiple_of(us, 128), 128)],
                    ubuf_v.at[slot * 4 + j], sem))
                vs = ivec[4 * rnd + j] & -128
                copies.append(pltpu.async_copy(
                    vt_hbm.at[:, pl.ds(pl.multiple_of(vs, 128), 128)],
                    vbuf_v.at[slot * 4 + j], sem))
            for cp in copies:
                cp.wait()

            bvec = slot * 4 + lane_mod4
            lanesu = _perm(lu, 4 * rnd + lane_mod4)
            lanesv = _perm(lv, 4 * rnd + lane_mod4)
            acc = jnp.zeros((L,), jnp.float32)
            for t in range(8):
                rvec = 4 * t + row_div4
                ug = plsc.load_gather(ubuf_v, [bvec, rvec, lanesu])
                vg = plsc.load_gather(vbuf_v, [bvec, rvec, lanesv])
                acc = acc + ug * vg
            acc = acc + _perm(acc, (iota + 8) & 15)
            acc = acc + _perm(acc, (iota + 4) & 15)
            placed = _perm(acc, (iota - 4 * rnd) & 15)
            mask = (iota >= 4 * rnd) & (iota < 4 * rnd + 4)
            out16 = jnp.where(mask, placed, out16)

        out_v[pl.ds(i * L, L)] = out16
        return carry

    lax.fori_loop(0, NITER, step, 0)

    pltpu.sync_copy(out_v, out_hbm.at[pl.ds(base, BPW)])


@jax.jit
def _cf(user, item, user_factors, item_factors):
    kern = pl.kernel(
        _cf_body,
        out_type=jax.ShapeDtypeStruct((B,), jnp.float32),
        mesh=plsc.VectorSubcoreMesh(core_axis_name="c", subcore_axis_name="s"),
        compiler_params=pltpu.CompilerParams(needs_layout_passes=False),
        scratch_types=[
            pltpu.VMEM((BPW,), jnp.int32),
            pltpu.VMEM((BPW,), jnp.int32),
            pltpu.VMEM((8, F, 128), jnp.float32),
            pltpu.VMEM((8, F, 128), jnp.float32),
            pltpu.VMEM((BPW,), jnp.float32),
            pltpu.SemaphoreType.DMA,
        ],
    )
    return kern(user_factors.T, item_factors.T, user, item)


def kernel(user, item, user_factors, item_factors):
    return _cf(user.astype(jnp.int32), item.astype(jnp.int32),
               user_factors, item_factors)


# speedup vs baseline: 4.6260x; 1.4723x over previous
import jax
import jax.numpy as jnp
from jax import lax
from jax.experimental import pallas as pl
from jax.experimental.pallas import tpu as pltpu
from jax.experimental.pallas import tpu_sc as plsc

B = 16384
F = 32
NUM_CORES = 2
NUM_SUBCORES = 16
NW = NUM_CORES * NUM_SUBCORES
BPW = B // NW
L = 16
NITER = BPW // L


def _iota():
    return lax.iota(jnp.int32, L)


def _perm(x, idx):
    return jnp.take(x, idx, axis=0)


def _cf_body(ut_hbm, vt_hbm, user_hbm, item_hbm, out_hbm,
             uidx_v, iidx_v, ubuf_v, vbuf_v, out_v, sem):
    wid = lax.axis_index("s") * NUM_CORES + lax.axis_index("c")
    base = wid * BPW

    pltpu.sync_copy(user_hbm.at[pl.ds(base, BPW)], uidx_v)
    pltpu.sync_copy(item_hbm.at[pl.ds(base, BPW)], iidx_v)

    iota = _iota()
    lane_mod4 = iota & 3
    row_div4 = iota >> 2

    def issue(uvec, ivec, rnd, slot):
        for j in range(4):
            us = uvec[4 * rnd + j] & -128
            pltpu.async_copy(
                ut_hbm.at[:, pl.ds(pl.multiple_of(us, 128), 128)],
                ubuf_v.at[slot * 4 + j], sem)
            vs = ivec[4 * rnd + j] & -128
            pltpu.async_copy(
                vt_hbm.at[:, pl.ds(pl.multiple_of(vs, 128), 128)],
                vbuf_v.at[slot * 4 + j], sem)

    def drain(slot):
        for j in range(4):
            pltpu.make_async_copy(
                ut_hbm.at[:, pl.ds(0, 128)], ubuf_v.at[slot * 4 + j], sem
            ).wait()
            pltpu.make_async_copy(
                vt_hbm.at[:, pl.ds(0, 128)], vbuf_v.at[slot * 4 + j], sem
            ).wait()

    def step(i, carry):
        uvec, ivec = carry
        lu = uvec & 127
        lv = ivec & 127
        nxt = jnp.minimum(i + 1, NITER - 1)
        unext = uidx_v[pl.ds(nxt * L, L)]
        inext = iidx_v[pl.ds(nxt * L, L)]
        out16 = jnp.zeros((L,), jnp.float32)

        for rnd in range(4):
            slot = rnd % 2
            if rnd < 3:
                issue(uvec, ivec, rnd + 1, (rnd + 1) % 2)
            else:
                @pl.when(i < NITER - 1)
                def _():
                    issue(unext, inext, 0, 0)
            drain(slot)

            bvec = slot * 4 + lane_mod4
            lanesu = _perm(lu, 4 * rnd + lane_mod4)
            lanesv = _perm(lv, 4 * rnd + lane_mod4)
            acc = jnp.zeros((L,), jnp.float32)
            for t in range(8):
                rvec = 4 * t + row_div4
                ug = plsc.load_gather(ubuf_v, [bvec, rvec, lanesu])
                vg = plsc.load_gather(vbuf_v, [bvec, rvec, lanesv])
                acc = acc + ug * vg
            acc = acc + _perm(acc, (iota + 8) & 15)
            acc = acc + _perm(acc, (iota + 4) & 15)
            placed = _perm(acc, (iota - 4 * rnd) & 15)
            mask = (iota >= 4 * rnd) & (iota < 4 * rnd + 4)
            out16 = jnp.where(mask, placed, out16)

        out_v[pl.ds(i * L, L)] = out16
        return (unext, inext)

    uvec0 = uidx_v[pl.ds(0, L)]
    ivec0 = iidx_v[pl.ds(0, L)]
    issue(uvec0, ivec0, 0, 0)
    lax.fori_loop(0, NITER, step, (uvec0, ivec0))

    pltpu.sync_copy(out_v, out_hbm.at[pl.ds(base, BPW)])


@jax.jit
def _cf(user, item, user_factors, item_factors):
    kern = pl.kernel(
        _cf_body,
        out_type=jax.ShapeDtypeStruct((B,), jnp.float32),
        mesh=plsc.VectorSubcoreMesh(core_axis_name="c", subcore_axis_name="s"),
        compiler_params=pltpu.CompilerParams(needs_layout_passes=False),
        scratch_types=[
            pltpu.VMEM((BPW,), jnp.int32),
            pltpu.VMEM((BPW,), jnp.int32),
            pltpu.VMEM((8, F, 128), jnp.float32),
            pltpu.VMEM((8, F, 128), jnp.float32),
            pltpu.VMEM((BPW,), jnp.float32),
            pltpu.SemaphoreType.DMA,
        ],
    )
    return kern(user_factors.T, item_factors.T, user, item)


def kernel(user, item, user_factors, item_factors):
    return _cf(user.astype(jnp.int32), item.astype(jnp.int32),
               user_factors, item_factors)
